# baseline (speedup 1.0000x reference)
import functools

import jax
import jax.numpy as jnp
from jax import lax
from jax.experimental import pallas as pl
from jax.experimental.pallas import tpu as pltpu
from jax.experimental.pallas import tpu_sc as plsc

GS = 240

NC = 2
NSUB = 16
NW = NC * NSUB
CH = 128


def _make_sc_scatter(e_pad, n_pad, hnf, r_rows, passes):
    rs = 2352
    seg = 18816
    n_strips = seg // rs
    n_seg = e_pad // seg
    chunk = 48
    kg = chunk // 16
    jmax = seg // chunk - 1
    trash = r_rows

    def body(z_hbm, row_hbm, ef_hbm, agg_hbm,
             rs0_v, rs1_v, eid_v, idx0_v, idx1_v, ef0_v, ef1_v, slab_v,
             ss0, ss1, se0, se1):
        wid = lax.axis_index("s") * NC + lax.axis_index("c")
        iota = lax.iota(jnp.int32, 16)
        rot = [jnp.bitwise_and(iota + s, 15) for s in range(16)]
        rbufs = [(rs0_v, ss0), (rs1_v, ss1)]

        def pfill(i, c):
            eid_v[pl.ds(i * 16, 16)] = jnp.zeros((16,), jnp.int32)
            return c
        lax.fori_loop(0, seg // 16, pfill, 0)

        def one_pass(p, c0):
            start = (p * NW + wid) * r_rows
            pltpu.sync_copy(z_hbm, slab_v.at[pl.ds(0, r_rows)])

            def one_seg(si, c1):
                sbase = si * seg

                cv = jnp.zeros((16,), jnp.int32)
                pltpu.async_copy(row_hbm.at[pl.ds(sbase, rs)], rs0_v, ss0)
                for st in range(n_strips):
                    buf, sem = rbufs[st % 2]
                    if st + 1 < n_strips:
                        nbuf, nsem = rbufs[(st + 1) % 2]
                        pltpu.async_copy(
                            row_hbm.at[pl.ds(sbase + (st + 1) * rs, rs)],
                            nbuf, nsem)
                    pltpu.make_async_copy(
                        row_hbm.at[pl.ds(sbase, rs)], buf, sem).wait()

                    def scan_vec(g, cv2, _buf=buf, _st=st):
                        r = _buf[pl.ds(g * 16, 16)]
                        m = (r >= start) & (r < start + r_rows)
                        eidv = sbase + _st * rs + g * 16 + iota
                        pk = eidv * 512 + (r - start)
                        pos = cv2 * 16 + iota
                        plsc.store_scatter(eid_v, [pos], pk, mask=m)
                        return cv2 + jnp.where(m, jnp.int32(1), jnp.int32(0))

                    cv = lax.fori_loop(0, rs // 16, scan_vec, cv)
                c_vec = cv
                mx = c_vec[0]
                for l in range(1, 16):
                    mx = jnp.maximum(mx, c_vec[l])

                def unpack(j, idx_b):
                    jc = jnp.minimum(j, jmax)
                    for kk in range(kg):
                        pv = eid_v[pl.ds(jc * chunk + kk * 16, 16)]
                        idx_b[pl.ds(kk * 16, 16)] = lax.shift_right_logical(
                            pv, 9)
                    return jc

                def acc(j, jc, ef_b):
                    def acc_group(gg, c3):
                        pv = eid_v[pl.ds(jc * chunk + gg * 16, 16)]
                        d = jnp.bitwise_and(pv, 511)
                        valid = (j * kg + gg) < c_vec
                        d = jnp.where(valid, d, jnp.int32(trash))
                        rows16 = gg * 16 + iota

                        def col_loop(gc, c4):
                            for s in range(16):
                                colv = gc * 16 + rot[s]
                                vals = plsc.load_gather(ef_b, [rows16, colv])
                                plsc.addupdate_scatter(slab_v, [d, colv],
                                                       vals)
                            return c4
                        lax.fori_loop(0, hnf // 16, col_loop, 0)
                        return c3
                    lax.fori_loop(0, kg, acc_group, 0)

                def pair(jp, c2):
                    j0 = jp * 2
                    jc0 = unpack(j0, idx0_v)
                    jc1 = unpack(j0 + 1, idx1_v)
                    cpa = pltpu.async_copy(ef_hbm.at[idx0_v], ef0_v, se0)
                    cpb = pltpu.async_copy(ef_hbm.at[idx1_v], ef1_v, se1)
                    cpa.wait()
                    acc(j0, jc0, ef0_v)
                    cpb.wait()
                    acc(j0 + 1, jc1, ef1_v)
                    return c2

                nch = (mx + kg - 1) // kg
                lax.fori_loop(0, (nch + 1) // 2, pair, 0)
                return c1

            lax.fori_loop(0, n_seg, one_seg, 0)
            pltpu.sync_copy(slab_v.at[pl.ds(0, r_rows)],
                            agg_hbm.at[pl.ds(start, r_rows)])
            return c0

        lax.fori_loop(0, passes, one_pass, 0)

    mesh = plsc.VectorSubcoreMesh(core_axis_name="c", subcore_axis_name="s",
                                  num_cores=NC, num_subcores=NSUB)
    return pl.kernel(
        body,
        out_type=jax.ShapeDtypeStruct((n_pad, hnf), jnp.float32),
        mesh=mesh,
        compiler_params=pltpu.CompilerParams(needs_layout_passes=False),
        scratch_types=[
            pltpu.VMEM((rs,), jnp.int32),
            pltpu.VMEM((rs,), jnp.int32),
            pltpu.VMEM((seg,), jnp.int32),
            pltpu.VMEM((chunk,), jnp.int32),
            pltpu.VMEM((chunk,), jnp.int32),
            pltpu.VMEM((chunk, hnf), jnp.float32),
            pltpu.VMEM((chunk, hnf), jnp.float32),
            pltpu.VMEM((r_rows + 8, hnf), jnp.float32),
            pltpu.SemaphoreType.DMA,
            pltpu.SemaphoreType.DMA,
            pltpu.SemaphoreType.DMA,
            pltpu.SemaphoreType.DMA,
        ],
    )


def _xsxt_body(x_ref, ws_ref, wt_ref, os_ref, ot_ref):
    os_ref[...] = jnp.dot(x_ref[...], ws_ref[...],
                          preferred_element_type=jnp.float32)
    ot_ref[...] = jnp.dot(x_ref[...], wt_ref[...],
                          preferred_element_type=jnp.float32)


def _make_sc_gather(e_pad, hnf):
    ch = 64
    per_w = e_pad // NW
    npair = per_w // (2 * ch)

    def body(xs_hbm, xt_hbm, row_hbm, col_hbm, out_hbm,
             ridx_v, cidx_v, a0_v, b0_v, a1_v, b1_v,
             sa0, sb0, sa1, sb1, sw0, sw1):
        wid = lax.axis_index("s") * NC + lax.axis_index("c")
        base = wid * per_w

        def add_rows(a_v, b_v):
            def add_row(r, c2):
                for c in range(hnf // 16):
                    sl = pl.ds(c * 16, 16)
                    a_v[r, sl] = a_v[r, sl] + b_v[r, sl]
                return c2
            lax.fori_loop(0, ch, add_row, 0)

        def pair(cp, carry):
            off = base + cp * 2 * ch
            pltpu.sync_copy(row_hbm.at[pl.ds(off, 2 * ch)], ridx_v)
            pltpu.sync_copy(col_hbm.at[pl.ds(off, 2 * ch)], cidx_v)
            cpa0 = pltpu.async_copy(
                xs_hbm.at[ridx_v.at[pl.ds(0, ch)]], a0_v, sa0)
            cpb0 = pltpu.async_copy(
                xt_hbm.at[cidx_v.at[pl.ds(0, ch)]], b0_v, sb0)
            cpa1 = pltpu.async_copy(
                xs_hbm.at[ridx_v.at[pl.ds(ch, ch)]], a1_v, sa1)
            cpb1 = pltpu.async_copy(
                xt_hbm.at[cidx_v.at[pl.ds(ch, ch)]], b1_v, sb1)
            cpa0.wait()
            cpb0.wait()
            add_rows(a0_v, b0_v)
            cw0 = pltpu.async_copy(a0_v, out_hbm.at[pl.ds(off, ch)], sw0)
            cpa1.wait()
            cpb1.wait()
            add_rows(a1_v, b1_v)
            cw1 = pltpu.async_copy(a1_v, out_hbm.at[pl.ds(off + ch, ch)], sw1)
            cw0.wait()
            cw1.wait()
            return carry

        lax.fori_loop(0, npair, pair, 0)

    mesh = plsc.VectorSubcoreMesh(core_axis_name="c", subcore_axis_name="s",
                                  num_cores=NC, num_subcores=NSUB)
    return pl.kernel(
        body,
        out_type=jax.ShapeDtypeStruct((e_pad, hnf), jnp.float32),
        mesh=mesh,
        scratch_types=[
            pltpu.VMEM((2 * ch,), jnp.int32),
            pltpu.VMEM((2 * ch,), jnp.int32),
            pltpu.VMEM((ch, hnf), jnp.float32),
            pltpu.VMEM((ch, hnf), jnp.float32),
            pltpu.VMEM((ch, hnf), jnp.float32),
            pltpu.VMEM((ch, hnf), jnp.float32),
            pltpu.SemaphoreType.DMA,
            pltpu.SemaphoreType.DMA,
            pltpu.SemaphoreType.DMA,
            pltpu.SemaphoreType.DMA,
            pltpu.SemaphoreType.DMA,
            pltpu.SemaphoreType.DMA,
        ],
    )


def _edge_body(s_ref, ea_ref, we_ref, eb1_ref, ew2_ref, eb2_ref, ng_ref,
               nb_ref, o_ref):
    h = s_ref[...] + jnp.dot(ea_ref[...], we_ref[...],
                             preferred_element_type=jnp.float32) + eb1_ref[...]
    h = jnp.maximum(h, 0.0)
    h = jnp.dot(h, ew2_ref[...], preferred_element_type=jnp.float32) + eb2_ref[...]
    h = jnp.maximum(h, 0.0)
    mu = jnp.mean(h, axis=-1, keepdims=True)
    d = h - mu
    var = jnp.mean(d * d, axis=-1, keepdims=True)
    o_ref[...] = d * jax.lax.rsqrt(var + 1e-5) * ng_ref[...] + nb_ref[...]


def _node_body(x_ref, agg_ref, wx_ref, wa_ref, wl_ref, nb1_ref, nw2_ref,
               nb2_ref, o_ref):
    agg = agg_ref[...]
    x = x_ref[...]
    lat = jnp.mean(agg, axis=0, keepdims=True)
    h = (jnp.dot(x, wx_ref[...], preferred_element_type=jnp.float32)
         + jnp.dot(agg, wa_ref[...], preferred_element_type=jnp.float32)
         + jnp.dot(lat, wl_ref[...], preferred_element_type=jnp.float32)
         + nb1_ref[...])
    h = jnp.maximum(h, 0.0)
    o = jnp.dot(h, nw2_ref[...], preferred_element_type=jnp.float32) + nb2_ref[...]
    o_ref[...] = o + x


def kernel(x, edge_index, edge_attr, ew1, eb1, ew2, eb2, ng, nb,
           nw1, nb1, nw2, nb2):
    n, inf = x.shape
    e, ein = edge_attr.shape
    hnf = ew2.shape[1]
    onf = nw2.shape[1]
    row = edge_index[0]
    col = edge_index[1]

    we = ew1[2 * inf:]
    wx = nw1[:inf]
    wa = nw1[inf:inf + hnf]
    wl = nw1[inf + hnf:]

    eb1r = eb1.reshape(1, hnf)
    eb2r = eb2.reshape(1, hnf)
    ngr = ng.reshape(1, hnf)
    nbr = nb.reshape(1, hnf)
    nb1r = nb1.reshape(1, hnf)
    nb2r = nb2.reshape(1, onf)

    xs, xt = pl.pallas_call(
        _xsxt_body,
        grid=(n // GS,),
        in_specs=[pl.BlockSpec((GS, inf), lambda i: (i, 0)),
                  pl.BlockSpec((inf, hnf), lambda i: (0, 0)),
                  pl.BlockSpec((inf, hnf), lambda i: (0, 0))],
        out_specs=[pl.BlockSpec((GS, hnf), lambda i: (i, 0)),
                   pl.BlockSpec((GS, hnf), lambda i: (i, 0))],
        out_shape=[jax.ShapeDtypeStruct((n, hnf), jnp.float32),
                   jax.ShapeDtypeStruct((n, hnf), jnp.float32)],
    )(x, ew1[:inf], ew1[inf:2 * inf])

    e_pad = -(-e // (NW * CH)) * (NW * CH)
    row_pad = jnp.pad(row, (0, e_pad - e))
    col_pad = jnp.pad(col, (0, e_pad - e))
    s = _make_sc_gather(e_pad, hnf)(xs, xt, row_pad, col_pad)

    be = 2000
    bcast = lambda i: (0, 0)
    edge_feat = pl.pallas_call(
        _edge_body,
        grid=(e // be,),
        in_specs=[pl.BlockSpec((be, hnf), lambda i: (i, 0)),
                  pl.BlockSpec((be, ein), lambda i: (i, 0)),
                  pl.BlockSpec((ein, hnf), bcast),
                  pl.BlockSpec((1, hnf), bcast),
                  pl.BlockSpec((hnf, hnf), bcast),
                  pl.BlockSpec((1, hnf), bcast),
                  pl.BlockSpec((1, hnf), bcast),
                  pl.BlockSpec((1, hnf), bcast)],
        out_specs=pl.BlockSpec((be, hnf), lambda i: (i, 0)),
        out_shape=jax.ShapeDtypeStruct((e, hnf), jnp.float32),
    )(s, edge_attr, we, eb1r, ew2, eb2r, ngr, nbr)

    r_rows, passes = 304, 3
    n_pad = NW * r_rows * passes
    e_pad2 = -(-e // 18816) * 18816
    row_pad2 = jnp.pad(row, (0, e_pad2 - e), constant_values=jnp.int32(1 << 30))
    zsrc = jnp.zeros((r_rows, hnf), jnp.float32)
    agg = _make_sc_scatter(e_pad2, n_pad, hnf, r_rows, passes)(
        zsrc, row_pad2, edge_feat)

    out = pl.pallas_call(
        _node_body,
        grid=(n // GS,),
        in_specs=[pl.BlockSpec((GS, inf), lambda i: (i, 0)),
                  pl.BlockSpec((GS, hnf), lambda i: (i, 0)),
                  pl.BlockSpec((inf, hnf), bcast),
                  pl.BlockSpec((hnf, hnf), bcast),
                  pl.BlockSpec((hnf, hnf), bcast),
                  pl.BlockSpec((1, hnf), bcast),
                  pl.BlockSpec((hnf, onf), bcast),
                  pl.BlockSpec((1, onf), bcast)],
        out_specs=pl.BlockSpec((GS, onf), lambda i: (i, 0)),
        out_shape=jax.ShapeDtypeStruct((n, onf), jnp.float32),
    )(x, agg, wx, wa, wl, nb1r, nw2, nb2r)

    return (out, edge_feat)


# speedup vs baseline: 1.0834x; 1.0834x over previous
import functools

import jax
import jax.numpy as jnp
from jax import lax
from jax.experimental import pallas as pl
from jax.experimental.pallas import tpu as pltpu
from jax.experimental.pallas import tpu_sc as plsc

GS = 240

NC = 2
NSUB = 16
NW = NC * NSUB
CH = 128


def _make_sc_scatter(e_pad, n_pad, hnf, r_rows, passes):
    rs = 2352
    seg = 18816
    n_strips = seg // rs
    n_seg = e_pad // seg
    chunk = 80
    kg = chunk // 16
    jmax = seg // chunk - 1
    trash = r_rows

    def body(z_hbm, row_hbm, ef_hbm, agg_hbm,
             rs0_v, rs1_v, eid_v, idx0_v, ef0_v, slab_v,
             ss0, ss1, se0):
        wid = lax.axis_index("s") * NC + lax.axis_index("c")
        iota = lax.iota(jnp.int32, 16)
        rot = [jnp.bitwise_and(iota + s, 15) for s in range(16)]
        rbufs = [(rs0_v, ss0), (rs1_v, ss1)]

        def pfill(i, c):
            eid_v[pl.ds(i * 16, 16)] = jnp.zeros((16,), jnp.int32)
            return c
        lax.fori_loop(0, seg // 16, pfill, 0)

        def one_pass(p, c0):
            start = (p * NW + wid) * r_rows
            pltpu.sync_copy(z_hbm, slab_v.at[pl.ds(0, r_rows)])

            def one_seg(si, c1):
                sbase = si * seg

                cv = jnp.zeros((16,), jnp.int32)
                pltpu.async_copy(row_hbm.at[pl.ds(sbase, rs)], rs0_v, ss0)
                for st in range(n_strips):
                    buf, sem = rbufs[st % 2]
                    if st + 1 < n_strips:
                        nbuf, nsem = rbufs[(st + 1) % 2]
                        pltpu.async_copy(
                            row_hbm.at[pl.ds(sbase + (st + 1) * rs, rs)],
                            nbuf, nsem)
                    pltpu.make_async_copy(
                        row_hbm.at[pl.ds(sbase, rs)], buf, sem).wait()

                    def scan_vec(g, cv2, _buf=buf, _st=st):
                        r = _buf[pl.ds(g * 16, 16)]
                        m = (r >= start) & (r < start + r_rows)
                        eidv = sbase + _st * rs + g * 16 + iota
                        pk = eidv * 512 + (r - start)
                        pos = cv2 * 16 + iota
                        plsc.store_scatter(eid_v, [pos], pk, mask=m)
                        return cv2 + jnp.where(m, jnp.int32(1), jnp.int32(0))

                    cv = lax.fori_loop(0, rs // 16, scan_vec, cv)
                c_vec = cv
                mx = c_vec[0]
                for l in range(1, 16):
                    mx = jnp.maximum(mx, c_vec[l])

                def unpack(j, idx_b):
                    jc = jnp.minimum(j, jmax)
                    for kk in range(kg):
                        pv = eid_v[pl.ds(jc * chunk + kk * 16, 16)]
                        idx_b[pl.ds(kk * 16, 16)] = lax.shift_right_logical(
                            pv, 9)
                    return jc

                def acc(j, jc, ef_b):
                    def acc_group(gg, c3):
                        pv = eid_v[pl.ds(jc * chunk + gg * 16, 16)]
                        d = jnp.bitwise_and(pv, 511)
                        valid = (j * kg + gg) < c_vec
                        d = jnp.where(valid, d, jnp.int32(trash))
                        rows16 = gg * 16 + iota

                        def col_loop(gc, c4):
                            for s in range(16):
                                colv = gc * 16 + rot[s]
                                vals = plsc.load_gather(ef_b, [rows16, colv])
                                plsc.addupdate_scatter(slab_v, [d, colv],
                                                       vals)
                            return c4
                        lax.fori_loop(0, hnf // 16, col_loop, 0)
                        return c3
                    lax.fori_loop(0, kg, acc_group, 0)

                def ch_body(j, c2):
                    jc = unpack(j, idx0_v)
                    pltpu.async_copy(ef_hbm.at[idx0_v], ef0_v, se0).wait()
                    acc(j, jc, ef0_v)
                    return c2

                lax.fori_loop(0, (mx + kg - 1) // kg, ch_body, 0)
                return c1

            lax.fori_loop(0, n_seg, one_seg, 0)
            pltpu.sync_copy(slab_v.at[pl.ds(0, r_rows)],
                            agg_hbm.at[pl.ds(start, r_rows)])
            return c0

        lax.fori_loop(0, passes, one_pass, 0)

    mesh = plsc.VectorSubcoreMesh(core_axis_name="c", subcore_axis_name="s",
                                  num_cores=NC, num_subcores=NSUB)
    return pl.kernel(
        body,
        out_type=jax.ShapeDtypeStruct((n_pad, hnf), jnp.float32),
        mesh=mesh,
        compiler_params=pltpu.CompilerParams(needs_layout_passes=False),
        scratch_types=[
            pltpu.VMEM((rs,), jnp.int32),
            pltpu.VMEM((rs,), jnp.int32),
            pltpu.VMEM((seg,), jnp.int32),
            pltpu.VMEM((chunk,), jnp.int32),
            pltpu.VMEM((chunk, hnf), jnp.float32),
            pltpu.VMEM((r_rows + 8, hnf), jnp.float32),
            pltpu.SemaphoreType.DMA,
            pltpu.SemaphoreType.DMA,
            pltpu.SemaphoreType.DMA,
        ],
    )


def _xsxt_body(x_ref, ws_ref, wt_ref, os_ref, ot_ref):
    os_ref[...] = jnp.dot(x_ref[...], ws_ref[...],
                          preferred_element_type=jnp.float32)
    ot_ref[...] = jnp.dot(x_ref[...], wt_ref[...],
                          preferred_element_type=jnp.float32)


def _make_sc_gather(e_pad, hnf):
    per_w = e_pad // NW
    nch = per_w // CH

    def body(xs_hbm, xt_hbm, row_hbm, col_hbm, out_hbm,
             ridx_v, cidx_v, a_v, b_v, sem_a, sem_b):
        wid = lax.axis_index("s") * NC + lax.axis_index("c")
        base = wid * per_w

        def chunk(ci, carry):
            off = base + ci * CH
            pltpu.sync_copy(row_hbm.at[pl.ds(off, CH)], ridx_v)
            pltpu.sync_copy(col_hbm.at[pl.ds(off, CH)], cidx_v)
            cpa = pltpu.async_copy(xs_hbm.at[ridx_v], a_v, sem_a)
            cpb = pltpu.async_copy(xt_hbm.at[cidx_v], b_v, sem_b)
            cpa.wait()
            cpb.wait()

            def add_row(r, c2):
                for c in range(hnf // 16):
                    sl = pl.ds(c * 16, 16)
                    a_v[r, sl] = a_v[r, sl] + b_v[r, sl]
                return c2

            lax.fori_loop(0, CH, add_row, 0)
            pltpu.sync_copy(a_v, out_hbm.at[pl.ds(off, CH)])
            return carry

        lax.fori_loop(0, nch, chunk, 0)

    mesh = plsc.VectorSubcoreMesh(core_axis_name="c", subcore_axis_name="s",
                                  num_cores=NC, num_subcores=NSUB)
    return pl.kernel(
        body,
        out_type=jax.ShapeDtypeStruct((e_pad, hnf), jnp.float32),
        mesh=mesh,
        scratch_types=[
            pltpu.VMEM((CH,), jnp.int32),
            pltpu.VMEM((CH,), jnp.int32),
            pltpu.VMEM((CH, hnf), jnp.float32),
            pltpu.VMEM((CH, hnf), jnp.float32),
            pltpu.SemaphoreType.DMA,
            pltpu.SemaphoreType.DMA,
        ],
    )


def _edge_body(s_ref, ea_ref, we_ref, eb1_ref, ew2_ref, eb2_ref, ng_ref,
               nb_ref, o_ref):
    h = s_ref[...] + jnp.dot(ea_ref[...], we_ref[...],
                             preferred_element_type=jnp.float32) + eb1_ref[...]
    h = jnp.maximum(h, 0.0)
    h = jnp.dot(h, ew2_ref[...], preferred_element_type=jnp.float32) + eb2_ref[...]
    h = jnp.maximum(h, 0.0)
    mu = jnp.mean(h, axis=-1, keepdims=True)
    d = h - mu
    var = jnp.mean(d * d, axis=-1, keepdims=True)
    o_ref[...] = d * jax.lax.rsqrt(var + 1e-5) * ng_ref[...] + nb_ref[...]


def _node_body(x_ref, agg_ref, wx_ref, wa_ref, wl_ref, nb1_ref, nw2_ref,
               nb2_ref, o_ref):
    agg = agg_ref[...]
    x = x_ref[...]
    lat = jnp.mean(agg, axis=0, keepdims=True)
    h = (jnp.dot(x, wx_ref[...], preferred_element_type=jnp.float32)
         + jnp.dot(agg, wa_ref[...], preferred_element_type=jnp.float32)
         + jnp.dot(lat, wl_ref[...], preferred_element_type=jnp.float32)
         + nb1_ref[...])
    h = jnp.maximum(h, 0.0)
    o = jnp.dot(h, nw2_ref[...], preferred_element_type=jnp.float32) + nb2_ref[...]
    o_ref[...] = o + x


def kernel(x, edge_index, edge_attr, ew1, eb1, ew2, eb2, ng, nb,
           nw1, nb1, nw2, nb2):
    n, inf = x.shape
    e, ein = edge_attr.shape
    hnf = ew2.shape[1]
    onf = nw2.shape[1]
    row = edge_index[0]
    col = edge_index[1]

    we = ew1[2 * inf:]
    wx = nw1[:inf]
    wa = nw1[inf:inf + hnf]
    wl = nw1[inf + hnf:]

    eb1r = eb1.reshape(1, hnf)
    eb2r = eb2.reshape(1, hnf)
    ngr = ng.reshape(1, hnf)
    nbr = nb.reshape(1, hnf)
    nb1r = nb1.reshape(1, hnf)
    nb2r = nb2.reshape(1, onf)

    xs, xt = pl.pallas_call(
        _xsxt_body,
        grid=(n // GS,),
        in_specs=[pl.BlockSpec((GS, inf), lambda i: (i, 0)),
                  pl.BlockSpec((inf, hnf), lambda i: (0, 0)),
                  pl.BlockSpec((inf, hnf), lambda i: (0, 0))],
        out_specs=[pl.BlockSpec((GS, hnf), lambda i: (i, 0)),
                   pl.BlockSpec((GS, hnf), lambda i: (i, 0))],
        out_shape=[jax.ShapeDtypeStruct((n, hnf), jnp.float32),
                   jax.ShapeDtypeStruct((n, hnf), jnp.float32)],
    )(x, ew1[:inf], ew1[inf:2 * inf])

    e_pad = -(-e // (NW * CH)) * (NW * CH)
    row_pad = jnp.pad(row, (0, e_pad - e))
    col_pad = jnp.pad(col, (0, e_pad - e))
    s = _make_sc_gather(e_pad, hnf)(xs, xt, row_pad, col_pad)

    be = 2000
    bcast = lambda i: (0, 0)
    edge_feat = pl.pallas_call(
        _edge_body,
        grid=(e // be,),
        in_specs=[pl.BlockSpec((be, hnf), lambda i: (i, 0)),
                  pl.BlockSpec((be, ein), lambda i: (i, 0)),
                  pl.BlockSpec((ein, hnf), bcast),
                  pl.BlockSpec((1, hnf), bcast),
                  pl.BlockSpec((hnf, hnf), bcast),
                  pl.BlockSpec((1, hnf), bcast),
                  pl.BlockSpec((1, hnf), bcast),
                  pl.BlockSpec((1, hnf), bcast)],
        out_specs=pl.BlockSpec((be, hnf), lambda i: (i, 0)),
        out_shape=jax.ShapeDtypeStruct((e, hnf), jnp.float32),
    )(s, edge_attr, we, eb1r, ew2, eb2r, ngr, nbr)

    r_rows, passes = 304, 3
    n_pad = NW * r_rows * passes
    e_pad2 = -(-e // 18816) * 18816
    row_pad2 = jnp.pad(row, (0, e_pad2 - e), constant_values=jnp.int32(1 << 30))
    zsrc = jnp.zeros((r_rows, hnf), jnp.float32)
    agg = _make_sc_scatter(e_pad2, n_pad, hnf, r_rows, passes)(
        zsrc, row_pad2, edge_feat)

    out = pl.pallas_call(
        _node_body,
        grid=(n // GS,),
        in_specs=[pl.BlockSpec((GS, inf), lambda i: (i, 0)),
                  pl.BlockSpec((GS, hnf), lambda i: (i, 0)),
                  pl.BlockSpec((inf, hnf), bcast),
                  pl.BlockSpec((hnf, hnf), bcast),
                  pl.BlockSpec((hnf, hnf), bcast),
                  pl.BlockSpec((1, hnf), bcast),
                  pl.BlockSpec((hnf, onf), bcast),
                  pl.BlockSpec((1, onf), bcast)],
        out_specs=pl.BlockSpec((GS, onf), lambda i: (i, 0)),
        out_shape=jax.ShapeDtypeStruct((n, onf), jnp.float32),
    )(x, agg, wx, wa, wl, nb1r, nw2, nb2r)

    return (out, edge_feat)


# speedup vs baseline: 1.1355x; 1.0481x over previous
import functools

import jax
import jax.numpy as jnp
from jax import lax
from jax.experimental import pallas as pl
from jax.experimental.pallas import tpu as pltpu
from jax.experimental.pallas import tpu_sc as plsc

GS = 240

NC = 2
NSUB = 16
NW = NC * NSUB
CH = 128


def _make_sc_scatter(e_pad, n_pad, hnf, r_rows, passes):
    rs = 2352
    seg = 18816
    n_strips = seg // rs
    n_seg = e_pad // seg
    chunk = 64
    kg = chunk // 16
    jmax = seg // chunk - 1
    trash = r_rows

    def body(z_hbm, row_hbm, ef_hbm, agg_hbm,
             rs0_v, rs1_v, eid_v, idx0_v, ef0_v, slab_v,
             ss0, ss1, se0):
        wid = lax.axis_index("s") * NC + lax.axis_index("c")
        iota = lax.iota(jnp.int32, 16)
        rot = [jnp.bitwise_and(iota + s, 15) for s in range(16)]
        rbufs = [(rs0_v, ss0), (rs1_v, ss1)]

        def pfill(i, c):
            eid_v[pl.ds(i * 16, 16)] = jnp.zeros((16,), jnp.int32)
            return c
        lax.fori_loop(0, seg // 16, pfill, 0)

        def one_pass(p, c0):
            start = (p * NW + wid) * r_rows
            pltpu.sync_copy(z_hbm, slab_v.at[pl.ds(0, r_rows)])

            def one_seg(si, c1):
                sbase = si * seg

                cv = jnp.zeros((16,), jnp.int32)
                pltpu.async_copy(row_hbm.at[pl.ds(sbase, rs)], rs0_v, ss0)
                for st in range(n_strips):
                    buf, sem = rbufs[st % 2]
                    if st + 1 < n_strips:
                        nbuf, nsem = rbufs[(st + 1) % 2]
                        pltpu.async_copy(
                            row_hbm.at[pl.ds(sbase + (st + 1) * rs, rs)],
                            nbuf, nsem)
                    pltpu.make_async_copy(
                        row_hbm.at[pl.ds(sbase, rs)], buf, sem).wait()

                    def scan_vec(g, cv2, _buf=buf, _st=st):
                        r = _buf[pl.ds(g * 16, 16)]
                        m = (r >= start) & (r < start + r_rows)
                        eidv = sbase + _st * rs + g * 16 + iota
                        pk = eidv * 512 + (r - start)
                        pos = cv2 * 16 + iota
                        plsc.store_scatter(eid_v, [pos], pk, mask=m)
                        return cv2 + jnp.where(m, jnp.int32(1), jnp.int32(0))

                    cv = lax.fori_loop(0, rs // 16, scan_vec, cv)
                c_vec = cv
                mx = c_vec[0]
                for l in range(1, 16):
                    mx = jnp.maximum(mx, c_vec[l])

                def unpack(j, idx_b):
                    jc = jnp.minimum(j, jmax)
                    for kk in range(kg):
                        pv = eid_v[pl.ds(jc * chunk + kk * 16, 16)]
                        idx_b[pl.ds(kk * 16, 16)] = lax.shift_right_logical(
                            pv, 9)
                    return jc

                def acc(j, jc, ef_b):
                    def acc_group(gg, c3):
                        pv = eid_v[pl.ds(jc * chunk + gg * 16, 16)]
                        d = jnp.bitwise_and(pv, 511)
                        valid = (j * kg + gg) < c_vec
                        d = jnp.where(valid, d, jnp.int32(trash))
                        rows16 = gg * 16 + iota

                        def col_loop(gc, c4):
                            for s in range(16):
                                colv = gc * 16 + rot[s]
                                vals = plsc.load_gather(ef_b, [rows16, colv])
                                plsc.addupdate_scatter(slab_v, [d, colv],
                                                       vals)
                            return c4
                        lax.fori_loop(0, hnf // 16, col_loop, 0)
                        return c3
                    lax.fori_loop(0, kg, acc_group, 0)

                def ch_body(j, c2):
                    jc = unpack(j, idx0_v)
                    pltpu.async_copy(ef_hbm.at[idx0_v], ef0_v, se0).wait()
                    acc(j, jc, ef0_v)
                    return c2

                lax.fori_loop(0, (mx + kg - 1) // kg, ch_body, 0)
                return c1

            lax.fori_loop(0, n_seg, one_seg, 0)
            pltpu.sync_copy(slab_v.at[pl.ds(0, r_rows)],
                            agg_hbm.at[pl.ds(start, r_rows)])
            return c0

        lax.fori_loop(0, passes, one_pass, 0)

    mesh = plsc.VectorSubcoreMesh(core_axis_name="c", subcore_axis_name="s",
                                  num_cores=NC, num_subcores=NSUB)
    return pl.kernel(
        body,
        out_type=jax.ShapeDtypeStruct((n_pad, hnf), jnp.float32),
        mesh=mesh,
        compiler_params=pltpu.CompilerParams(needs_layout_passes=False),
        scratch_types=[
            pltpu.VMEM((rs,), jnp.int32),
            pltpu.VMEM((rs,), jnp.int32),
            pltpu.VMEM((seg,), jnp.int32),
            pltpu.VMEM((chunk,), jnp.int32),
            pltpu.VMEM((chunk, hnf), jnp.float32),
            pltpu.VMEM((r_rows + 8, hnf), jnp.float32),
            pltpu.SemaphoreType.DMA,
            pltpu.SemaphoreType.DMA,
            pltpu.SemaphoreType.DMA,
        ],
    )


def _xsxt_body(x_ref, ws_ref, wt_ref, os_ref, ot_ref):
    os_ref[...] = jnp.dot(x_ref[...], ws_ref[...],
                          preferred_element_type=jnp.float32)
    ot_ref[...] = jnp.dot(x_ref[...], wt_ref[...],
                          preferred_element_type=jnp.float32)


def _make_sc_gather(e_pad, hnf):
    per_w = e_pad // NW
    nch = per_w // CH

    def body(xs_hbm, xt_hbm, row_hbm, col_hbm, out_hbm,
             ridx_v, cidx_v, a_v, b_v, sem_a, sem_b):
        wid = lax.axis_index("s") * NC + lax.axis_index("c")
        base = wid * per_w

        def chunk(ci, carry):
            off = base + ci * CH
            pltpu.sync_copy(row_hbm.at[pl.ds(off, CH)], ridx_v)
            pltpu.sync_copy(col_hbm.at[pl.ds(off, CH)], cidx_v)
            cpa = pltpu.async_copy(xs_hbm.at[ridx_v], a_v, sem_a)
            cpb = pltpu.async_copy(xt_hbm.at[cidx_v], b_v, sem_b)
            cpa.wait()
            cpb.wait()

            def add_row(r, c2):
                for c in range(hnf // 16):
                    sl = pl.ds(c * 16, 16)
                    a_v[r, sl] = a_v[r, sl] + b_v[r, sl]
                return c2

            lax.fori_loop(0, CH, add_row, 0)
            pltpu.sync_copy(a_v, out_hbm.at[pl.ds(off, CH)])
            return carry

        lax.fori_loop(0, nch, chunk, 0)

    mesh = plsc.VectorSubcoreMesh(core_axis_name="c", subcore_axis_name="s",
                                  num_cores=NC, num_subcores=NSUB)
    return pl.kernel(
        body,
        out_type=jax.ShapeDtypeStruct((e_pad, hnf), jnp.float32),
        mesh=mesh,
        scratch_types=[
            pltpu.VMEM((CH,), jnp.int32),
            pltpu.VMEM((CH,), jnp.int32),
            pltpu.VMEM((CH, hnf), jnp.float32),
            pltpu.VMEM((CH, hnf), jnp.float32),
            pltpu.SemaphoreType.DMA,
            pltpu.SemaphoreType.DMA,
        ],
    )


def _edge_body(s_ref, ea_ref, we_ref, eb1_ref, ew2_ref, eb2_ref, ng_ref,
               nb_ref, o_ref):
    h = s_ref[...] + jnp.dot(ea_ref[...], we_ref[...],
                             preferred_element_type=jnp.float32) + eb1_ref[...]
    h = jnp.maximum(h, 0.0)
    h = jnp.dot(h, ew2_ref[...], preferred_element_type=jnp.float32) + eb2_ref[...]
    h = jnp.maximum(h, 0.0)
    mu = jnp.mean(h, axis=-1, keepdims=True)
    d = h - mu
    var = jnp.mean(d * d, axis=-1, keepdims=True)
    o_ref[...] = d * jax.lax.rsqrt(var + 1e-5) * ng_ref[...] + nb_ref[...]


def _node_body(x_ref, agg_ref, wx_ref, wa_ref, wl_ref, nb1_ref, nw2_ref,
               nb2_ref, o_ref):
    agg = agg_ref[...]
    x = x_ref[...]
    lat = jnp.mean(agg, axis=0, keepdims=True)
    h = (jnp.dot(x, wx_ref[...], preferred_element_type=jnp.float32)
         + jnp.dot(agg, wa_ref[...], preferred_element_type=jnp.float32)
         + jnp.dot(lat, wl_ref[...], preferred_element_type=jnp.float32)
         + nb1_ref[...])
    h = jnp.maximum(h, 0.0)
    o = jnp.dot(h, nw2_ref[...], preferred_element_type=jnp.float32) + nb2_ref[...]
    o_ref[...] = o + x


def kernel(x, edge_index, edge_attr, ew1, eb1, ew2, eb2, ng, nb,
           nw1, nb1, nw2, nb2):
    n, inf = x.shape
    e, ein = edge_attr.shape
    hnf = ew2.shape[1]
    onf = nw2.shape[1]
    row = edge_index[0]
    col = edge_index[1]

    we = ew1[2 * inf:]
    wx = nw1[:inf]
    wa = nw1[inf:inf + hnf]
    wl = nw1[inf + hnf:]

    eb1r = eb1.reshape(1, hnf)
    eb2r = eb2.reshape(1, hnf)
    ngr = ng.reshape(1, hnf)
    nbr = nb.reshape(1, hnf)
    nb1r = nb1.reshape(1, hnf)
    nb2r = nb2.reshape(1, onf)

    xs, xt = pl.pallas_call(
        _xsxt_body,
        grid=(n // GS,),
        in_specs=[pl.BlockSpec((GS, inf), lambda i: (i, 0)),
                  pl.BlockSpec((inf, hnf), lambda i: (0, 0)),
                  pl.BlockSpec((inf, hnf), lambda i: (0, 0))],
        out_specs=[pl.BlockSpec((GS, hnf), lambda i: (i, 0)),
                   pl.BlockSpec((GS, hnf), lambda i: (i, 0))],
        out_shape=[jax.ShapeDtypeStruct((n, hnf), jnp.float32),
                   jax.ShapeDtypeStruct((n, hnf), jnp.float32)],
    )(x, ew1[:inf], ew1[inf:2 * inf])

    e_pad = -(-e // (NW * CH)) * (NW * CH)
    row_pad = jnp.pad(row, (0, e_pad - e))
    col_pad = jnp.pad(col, (0, e_pad - e))
    s = _make_sc_gather(e_pad, hnf)(xs, xt, row_pad, col_pad)

    be = 2000
    bcast = lambda i: (0, 0)
    edge_feat = pl.pallas_call(
        _edge_body,
        grid=(e // be,),
        in_specs=[pl.BlockSpec((be, hnf), lambda i: (i, 0)),
                  pl.BlockSpec((be, ein), lambda i: (i, 0)),
                  pl.BlockSpec((ein, hnf), bcast),
                  pl.BlockSpec((1, hnf), bcast),
                  pl.BlockSpec((hnf, hnf), bcast),
                  pl.BlockSpec((1, hnf), bcast),
                  pl.BlockSpec((1, hnf), bcast),
                  pl.BlockSpec((1, hnf), bcast)],
        out_specs=pl.BlockSpec((be, hnf), lambda i: (i, 0)),
        out_shape=jax.ShapeDtypeStruct((e, hnf), jnp.float32),
    )(s, edge_attr, we, eb1r, ew2, eb2r, ngr, nbr)

    r_rows, passes = 304, 3
    n_pad = NW * r_rows * passes
    e_pad2 = -(-e // 18816) * 18816
    row_pad2 = jnp.pad(row, (0, e_pad2 - e), constant_values=jnp.int32(1 << 30))
    zsrc = jnp.zeros((r_rows, hnf), jnp.float32)
    agg = _make_sc_scatter(e_pad2, n_pad, hnf, r_rows, passes)(
        zsrc, row_pad2, edge_feat)

    out = pl.pallas_call(
        _node_body,
        grid=(n // GS,),
        in_specs=[pl.BlockSpec((GS, inf), lambda i: (i, 0)),
                  pl.BlockSpec((GS, hnf), lambda i: (i, 0)),
                  pl.BlockSpec((inf, hnf), bcast),
                  pl.BlockSpec((hnf, hnf), bcast),
                  pl.BlockSpec((hnf, hnf), bcast),
                  pl.BlockSpec((1, hnf), bcast),
                  pl.BlockSpec((hnf, onf), bcast),
                  pl.BlockSpec((1, onf), bcast)],
        out_specs=pl.BlockSpec((GS, onf), lambda i: (i, 0)),
        out_shape=jax.ShapeDtypeStruct((n, onf), jnp.float32),
    )(x, agg, wx, wa, wl, nb1r, nw2, nb2r)

    return (out, edge_feat)


# speedup vs baseline: 1.1383x; 1.0024x over previous
import functools

import jax
import jax.numpy as jnp
from jax import lax
from jax.experimental import pallas as pl
from jax.experimental.pallas import tpu as pltpu
from jax.experimental.pallas import tpu_sc as plsc

GS = 240

NC = 2
NSUB = 16
NW = NC * NSUB
CH = 128


def _make_sc_scatter(e_pad, n_pad, hnf, r_rows, passes):
    rs = 2352
    seg = 18816
    n_strips = seg // rs
    n_seg = e_pad // seg
    chunk = 64
    kg = chunk // 16
    jmax = seg // chunk - 1
    trash = r_rows

    def body(z_hbm, row_hbm, ef_hbm, agg_hbm,
             rs0_v, rs1_v, eid_v, idx0_v, ef0_v, slab_v,
             ss0, ss1, se0):
        wid = lax.axis_index("s") * NC + lax.axis_index("c")
        iota = lax.iota(jnp.int32, 16)
        rot = [jnp.bitwise_and(iota + s, 15) for s in range(16)]
        rbufs = [(rs0_v, ss0), (rs1_v, ss1)]

        def pfill(i, c):
            eid_v[pl.ds(i * 16, 16)] = jnp.zeros((16,), jnp.int32)
            return c
        lax.fori_loop(0, seg // 16, pfill, 0)

        def one_pass(p, c0):
            start = (p * NW + wid) * r_rows
            pltpu.sync_copy(z_hbm, slab_v.at[pl.ds(0, r_rows)])

            def one_seg(si, c1):
                sbase = si * seg

                cv = jnp.zeros((16,), jnp.int32)
                pltpu.async_copy(row_hbm.at[pl.ds(sbase, rs)], rs0_v, ss0)
                for st in range(n_strips):
                    buf, sem = rbufs[st % 2]
                    if st + 1 < n_strips:
                        nbuf, nsem = rbufs[(st + 1) % 2]
                        pltpu.async_copy(
                            row_hbm.at[pl.ds(sbase + (st + 1) * rs, rs)],
                            nbuf, nsem)
                    pltpu.make_async_copy(
                        row_hbm.at[pl.ds(sbase, rs)], buf, sem).wait()

                    def scan_vec(g, cv2, _buf=buf, _st=st):
                        r = _buf[pl.ds(g * 16, 16)]
                        m = (r >= start) & (r < start + r_rows)
                        eidv = sbase + _st * rs + g * 16 + iota
                        pk = eidv * 512 + (r - start)
                        pos = cv2 * 16 + iota
                        plsc.store_scatter(eid_v, [pos], pk, mask=m)
                        return cv2 + jnp.where(m, jnp.int32(1), jnp.int32(0))

                    cv = lax.fori_loop(0, rs // 16, scan_vec, cv)
                c_vec = cv
                mx = c_vec[0]
                for l in range(1, 16):
                    mx = jnp.maximum(mx, c_vec[l])

                def unpack(j, idx_b):
                    jc = jnp.minimum(j, jmax)
                    for kk in range(kg):
                        pv = eid_v[pl.ds(jc * chunk + kk * 16, 16)]
                        idx_b[pl.ds(kk * 16, 16)] = lax.shift_right_logical(
                            pv, 9)
                    return jc

                def acc(j, jc, ef_b):
                    def acc_group(gg, c3):
                        pv = eid_v[pl.ds(jc * chunk + gg * 16, 16)]
                        d = jnp.bitwise_and(pv, 511)
                        valid = (j * kg + gg) < c_vec
                        d = jnp.where(valid, d, jnp.int32(trash))
                        rows16 = gg * 16 + iota

                        def col_loop(gc, c4):
                            for s in range(16):
                                colv = gc * 16 + rot[s]
                                vals = plsc.load_gather(ef_b, [rows16, colv])
                                plsc.addupdate_scatter(slab_v, [d, colv],
                                                       vals)
                            return c4
                        lax.fori_loop(0, hnf // 16, col_loop, 0)
                        return c3
                    lax.fori_loop(0, kg, acc_group, 0)

                def ch_body(j, c2):
                    jc = unpack(j, idx0_v)
                    pltpu.async_copy(ef_hbm.at[idx0_v], ef0_v, se0).wait()
                    acc(j, jc, ef0_v)
                    return c2

                lax.fori_loop(0, (mx + kg - 1) // kg, ch_body, 0)
                return c1

            lax.fori_loop(0, n_seg, one_seg, 0)
            pltpu.sync_copy(slab_v.at[pl.ds(0, r_rows)],
                            agg_hbm.at[pl.ds(start, r_rows)])
            return c0

        lax.fori_loop(0, passes, one_pass, 0)

    mesh = plsc.VectorSubcoreMesh(core_axis_name="c", subcore_axis_name="s",
                                  num_cores=NC, num_subcores=NSUB)
    return pl.kernel(
        body,
        out_type=jax.ShapeDtypeStruct((n_pad, hnf), jnp.float32),
        mesh=mesh,
        compiler_params=pltpu.CompilerParams(needs_layout_passes=False),
        scratch_types=[
            pltpu.VMEM((rs,), jnp.int32),
            pltpu.VMEM((rs,), jnp.int32),
            pltpu.VMEM((seg,), jnp.int32),
            pltpu.VMEM((chunk,), jnp.int32),
            pltpu.VMEM((chunk, hnf), jnp.float32),
            pltpu.VMEM((r_rows + 8, hnf), jnp.float32),
            pltpu.SemaphoreType.DMA,
            pltpu.SemaphoreType.DMA,
            pltpu.SemaphoreType.DMA,
        ],
    )


def _xsxt_body(x_ref, ws_ref, wt_ref, os_ref, ot_ref):
    os_ref[...] = jnp.dot(x_ref[...], ws_ref[...],
                          preferred_element_type=jnp.float32)
    ot_ref[...] = jnp.dot(x_ref[...], wt_ref[...],
                          preferred_element_type=jnp.float32)


def _make_sc_gather(e_pad, hnf):
    per_w = e_pad // NW
    nch = per_w // CH

    def body(xs_hbm, xt_hbm, row_hbm, col_hbm, out_hbm,
             ridx_v, cidx_v, a_v, b_v, sem_a, sem_b):
        wid = lax.axis_index("s") * NC + lax.axis_index("c")
        base = wid * per_w

        def chunk(ci, carry):
            off = base + ci * CH
            pltpu.sync_copy(row_hbm.at[pl.ds(off, CH)], ridx_v)
            pltpu.sync_copy(col_hbm.at[pl.ds(off, CH)], cidx_v)
            cpa = pltpu.async_copy(xs_hbm.at[ridx_v], a_v, sem_a)
            cpb = pltpu.async_copy(xt_hbm.at[cidx_v], b_v, sem_b)
            cpa.wait()
            cpb.wait()

            def add_row(r, c2):
                for c in range(hnf // 16):
                    sl = pl.ds(c * 16, 16)
                    a_v[r, sl] = a_v[r, sl] + b_v[r, sl]
                return c2

            lax.fori_loop(0, CH, add_row, 0)
            pltpu.sync_copy(a_v, out_hbm.at[pl.ds(off, CH)])
            return carry

        lax.fori_loop(0, nch, chunk, 0)

    mesh = plsc.VectorSubcoreMesh(core_axis_name="c", subcore_axis_name="s",
                                  num_cores=NC, num_subcores=NSUB)
    return pl.kernel(
        body,
        out_type=jax.ShapeDtypeStruct((e_pad, hnf), jnp.float32),
        mesh=mesh,
        scratch_types=[
            pltpu.VMEM((CH,), jnp.int32),
            pltpu.VMEM((CH,), jnp.int32),
            pltpu.VMEM((CH, hnf), jnp.float32),
            pltpu.VMEM((CH, hnf), jnp.float32),
            pltpu.SemaphoreType.DMA,
            pltpu.SemaphoreType.DMA,
        ],
    )


def _edge_body(s_ref, ea_ref, we_ref, eb1_ref, ew2_ref, eb2_ref, ng_ref,
               nb_ref, o_ref):
    h = s_ref[...] + jnp.dot(ea_ref[...], we_ref[...],
                             preferred_element_type=jnp.float32) + eb1_ref[...]
    h = jnp.maximum(h, 0.0)
    h = jnp.dot(h.astype(jnp.bfloat16), ew2_ref[...].astype(jnp.bfloat16),
                preferred_element_type=jnp.float32) + eb2_ref[...]
    h = jnp.maximum(h, 0.0)
    mu = jnp.mean(h, axis=-1, keepdims=True)
    d = h - mu
    var = jnp.mean(d * d, axis=-1, keepdims=True)
    o_ref[...] = d * jax.lax.rsqrt(var + 1e-5) * ng_ref[...] + nb_ref[...]


def _node_body(x_ref, agg_ref, wx_ref, wa_ref, wl_ref, nb1_ref, nw2_ref,
               nb2_ref, o_ref):
    agg = agg_ref[...]
    x = x_ref[...]
    lat = jnp.mean(agg, axis=0, keepdims=True)
    h = (jnp.dot(x, wx_ref[...], preferred_element_type=jnp.float32)
         + jnp.dot(agg, wa_ref[...], preferred_element_type=jnp.float32)
         + jnp.dot(lat, wl_ref[...], preferred_element_type=jnp.float32)
         + nb1_ref[...])
    h = jnp.maximum(h, 0.0)
    o = jnp.dot(h, nw2_ref[...], preferred_element_type=jnp.float32) + nb2_ref[...]
    o_ref[...] = o + x


def kernel(x, edge_index, edge_attr, ew1, eb1, ew2, eb2, ng, nb,
           nw1, nb1, nw2, nb2):
    n, inf = x.shape
    e, ein = edge_attr.shape
    hnf = ew2.shape[1]
    onf = nw2.shape[1]
    row = edge_index[0]
    col = edge_index[1]

    we = ew1[2 * inf:]
    wx = nw1[:inf]
    wa = nw1[inf:inf + hnf]
    wl = nw1[inf + hnf:]

    eb1r = eb1.reshape(1, hnf)
    eb2r = eb2.reshape(1, hnf)
    ngr = ng.reshape(1, hnf)
    nbr = nb.reshape(1, hnf)
    nb1r = nb1.reshape(1, hnf)
    nb2r = nb2.reshape(1, onf)

    xs, xt = pl.pallas_call(
        _xsxt_body,
        grid=(n // GS,),
        in_specs=[pl.BlockSpec((GS, inf), lambda i: (i, 0)),
                  pl.BlockSpec((inf, hnf), lambda i: (0, 0)),
                  pl.BlockSpec((inf, hnf), lambda i: (0, 0))],
        out_specs=[pl.BlockSpec((GS, hnf), lambda i: (i, 0)),
                   pl.BlockSpec((GS, hnf), lambda i: (i, 0))],
        out_shape=[jax.ShapeDtypeStruct((n, hnf), jnp.float32),
                   jax.ShapeDtypeStruct((n, hnf), jnp.float32)],
    )(x, ew1[:inf], ew1[inf:2 * inf])

    e_pad = -(-e // (NW * CH)) * (NW * CH)
    row_pad = jnp.pad(row, (0, e_pad - e))
    col_pad = jnp.pad(col, (0, e_pad - e))
    s = _make_sc_gather(e_pad, hnf)(xs, xt, row_pad, col_pad)

    be = 2000
    bcast = lambda i: (0, 0)
    edge_feat = pl.pallas_call(
        _edge_body,
        grid=(e // be,),
        in_specs=[pl.BlockSpec((be, hnf), lambda i: (i, 0)),
                  pl.BlockSpec((be, ein), lambda i: (i, 0)),
                  pl.BlockSpec((ein, hnf), bcast),
                  pl.BlockSpec((1, hnf), bcast),
                  pl.BlockSpec((hnf, hnf), bcast),
                  pl.BlockSpec((1, hnf), bcast),
                  pl.BlockSpec((1, hnf), bcast),
                  pl.BlockSpec((1, hnf), bcast)],
        out_specs=pl.BlockSpec((be, hnf), lambda i: (i, 0)),
        out_shape=jax.ShapeDtypeStruct((e, hnf), jnp.float32),
    )(s, edge_attr, we, eb1r, ew2, eb2r, ngr, nbr)

    r_rows, passes = 304, 3
    n_pad = NW * r_rows * passes
    e_pad2 = -(-e // 18816) * 18816
    row_pad2 = jnp.pad(row, (0, e_pad2 - e), constant_values=jnp.int32(1 << 30))
    zsrc = jnp.zeros((r_rows, hnf), jnp.float32)
    agg = _make_sc_scatter(e_pad2, n_pad, hnf, r_rows, passes)(
        zsrc, row_pad2, edge_feat)

    out = pl.pallas_call(
        _node_body,
        grid=(n // GS,),
        in_specs=[pl.BlockSpec((GS, inf), lambda i: (i, 0)),
                  pl.BlockSpec((GS, hnf), lambda i: (i, 0)),
                  pl.BlockSpec((inf, hnf), bcast),
                  pl.BlockSpec((hnf, hnf), bcast),
                  pl.BlockSpec((hnf, hnf), bcast),
                  pl.BlockSpec((1, hnf), bcast),
                  pl.BlockSpec((hnf, onf), bcast),
                  pl.BlockSpec((1, onf), bcast)],
        out_specs=pl.BlockSpec((GS, onf), lambda i: (i, 0)),
        out_shape=jax.ShapeDtypeStruct((n, onf), jnp.float32),
    )(x, agg, wx, wa, wl, nb1r, nw2, nb2r)

    return (out, edge_feat)
